# Initial kernel scaffold; baseline (speedup 1.0000x reference)
import jax
import jax.numpy as jnp
from jax.experimental import pallas as pl


def kernel(x_lig, x_pro, lig_edge_index, lig_edge_attr, pro_edge_index, pro_edge_attr, inter_edge_index, inter_edge_attr, lig_params, pro_params, ip_params):
    raise NotImplementedError("write your pallas kernel here")


# speedup vs baseline: 1.1254x; 1.1254x over previous
import functools

import jax
import jax.numpy as jnp
from jax import lax
from jax.experimental import pallas as pl
from jax.experimental.pallas import tpu as pltpu
from jax.experimental.pallas import tpu_sc as plsc

HID = 128
NLANE = 16
NCORE = 2
NSUB = 16
NW = NCORE * NSUB
EB = 128


def _dense_node_body(x_ref, wl_ref, bl_ref, wr_ref, br_ref,
                     xl0_ref, xl1_ref, xr_ref):
    x = x_ref[...]
    xl = jnp.dot(x, wl_ref[...], preferred_element_type=jnp.float32) + bl_ref[...]
    xr = jnp.dot(x, wr_ref[...], preferred_element_type=jnp.float32) + br_ref[...]
    xl0_ref[...] = xl[:, :64]
    xl1_ref[...] = xl[:, 64:]
    xr_ref[...] = xr


def _dense_node(x, wl, bl, wr, br):
    n = x.shape[0]
    br_blk = 256
    grid = (pl.cdiv(n, br_blk),)
    return pl.pallas_call(
        _dense_node_body,
        grid=grid,
        in_specs=[
            pl.BlockSpec((br_blk, HID), lambda i: (i, 0)),
            pl.BlockSpec((HID, HID), lambda i: (0, 0)),
            pl.BlockSpec((1, HID), lambda i: (0, 0)),
            pl.BlockSpec((HID, HID), lambda i: (0, 0)),
            pl.BlockSpec((1, HID), lambda i: (0, 0)),
        ],
        out_specs=[
            pl.BlockSpec((br_blk, 64), lambda i: (i, 0)),
            pl.BlockSpec((br_blk, 64), lambda i: (i, 0)),
            pl.BlockSpec((br_blk, HID), lambda i: (i, 0)),
        ],
        out_shape=[
            jax.ShapeDtypeStruct((n, 64), jnp.float32),
            jax.ShapeDtypeStruct((n, 64), jnp.float32),
            jax.ShapeDtypeStruct((n, HID), jnp.float32),
        ],
    )(x, wl, bl.reshape(1, HID), wr, br.reshape(1, HID))


def _dense_edge_body(a_ref, w_ref, o_ref):
    o_ref[...] = jnp.dot(a_ref[...], w_ref[...],
                         preferred_element_type=jnp.float32)


def _dense_edge(edge_attr, we):
    e, k = edge_attr.shape
    br_blk = 512
    return pl.pallas_call(
        _dense_edge_body,
        grid=(pl.cdiv(e, br_blk),),
        in_specs=[
            pl.BlockSpec((br_blk, k), lambda i: (i, 0)),
            pl.BlockSpec((k, HID), lambda i: (0, 0)),
        ],
        out_specs=pl.BlockSpec((br_blk, HID), lambda i: (i, 0)),
        out_shape=jax.ShapeDtypeStruct((e, HID), jnp.float32),
    )(edge_attr, we)


def _normalize_body(acc_ref, den_ref, bias_ref, out_ref):
    d = den_ref[0, :, 0:1] + den_ref[1, :, 0:1] + 1e-16
    out_ref[:, :64] = acc_ref[0] / d + bias_ref[:, :64]
    out_ref[:, 64:] = acc_ref[1] / d + bias_ref[:, 64:]


def _normalize(acc, den, bias, n):
    br_blk = 256
    return pl.pallas_call(
        _normalize_body,
        grid=(pl.cdiv(n, br_blk),),
        in_specs=[
            pl.BlockSpec((2, br_blk, 64), lambda i: (0, i, 0)),
            pl.BlockSpec((2, br_blk, NLANE), lambda i: (0, i, 0)),
            pl.BlockSpec((1, HID), lambda i: (0, 0)),
        ],
        out_specs=pl.BlockSpec((br_blk, HID), lambda i: (i, 0)),
        out_shape=jax.ShapeDtypeStruct((n, HID), jnp.float32),
    )(acc, den, bias.reshape(1, HID))


def _pad_nodes(n):
    return ((n + NSUB * 8 - 1) // (NSUB * 8)) * (NSUB * 8)


def _pass1_body(n_nodes, e_pad, e_real,
                xl0_hbm, xl1_hbm, xr_hbm, ea_hbm, src_hbm, dst_hbm, att_hbm,
                zeros_hbm,
                w_hbm, den_hbm,
                idxs_v, idxd_v, bufl0_v, bufl1_v, bufd_v, bufe_v,
                stage_v, w_v, att_v, den_sp, sem):
    c = lax.axis_index("c")
    s = lax.axis_index("s")
    wid = s * NCORE + c
    epw = e_pad // NW
    nblocks = epw // EB
    rows_per_sub = n_nodes // NSUB

    pltpu.sync_copy(zeros_hbm.at[pl.ds(s * rows_per_sub, rows_per_sub), :],
                    den_sp.at[pl.ds(s * rows_per_sub, rows_per_sub), :])
    pltpu.sync_copy(att_hbm, att_v)
    pltpu.sync_copy(zeros_hbm.at[pl.ds(0, EB), :], stage_v)
    plsc.subcore_barrier()

    def block(t, _):
        base = wid * epw + t * EB
        pltpu.sync_copy(src_hbm.at[pl.ds(base, EB)], idxs_v)
        pltpu.sync_copy(dst_hbm.at[pl.ds(base, EB)], idxd_v)
        pltpu.async_copy(xl0_hbm.at[idxs_v], bufl0_v, sem).wait()
        pltpu.async_copy(xl1_hbm.at[idxs_v], bufl1_v, sem).wait()
        pltpu.async_copy(xr_hbm.at[idxd_v], bufd_v, sem).wait()
        pltpu.sync_copy(ea_hbm.at[pl.ds(base, EB), :], bufe_v)

        def group(g, _):
            ei = lax.iota(jnp.int32, NLANE) + g * NLANE
            acc = jnp.zeros((NLANE,), jnp.float32)
            for f in range(HID):
                fi = jnp.full((NLANE,), f, jnp.int32)
                if f < 64:
                    a = plsc.load_gather(bufl0_v, [ei, fi])
                else:
                    a = plsc.load_gather(
                        bufl1_v, [ei, jnp.full((NLANE,), f - 64, jnp.int32)])
                v = (a + plsc.load_gather(bufd_v, [ei, fi])
                     + plsc.load_gather(bufe_v, [ei, fi]))
                v = jnp.where(v >= 0, v, 0.2 * v)
                acc = acc + att_v[f, :] * v
            w = jnp.exp(acc)
            gmask = (base + ei) < e_real
            w = jnp.where(gmask, w, 0.0)
            w_v[pl.ds(g * NLANE, NLANE)] = w
            plsc.store_scatter(stage_v,
                               [ei, jnp.zeros((NLANE,), jnp.int32)], w)
            return _

        lax.fori_loop(0, EB // NLANE, group, None)
        pltpu.sync_copy(w_v, w_hbm.at[pl.ds(base, EB)])
        pltpu.sync_copy(stage_v, den_sp.at[idxd_v], add=True)
        return _

    lax.fori_loop(0, nblocks, block, None)
    plsc.subcore_barrier()
    pltpu.sync_copy(den_sp.at[pl.ds(s * rows_per_sub, rows_per_sub), :],
                    den_hbm.at[c].at[pl.ds(s * rows_per_sub, rows_per_sub), :])


@functools.lru_cache(maxsize=None)
def _make_pass1(n_nodes, e_pad, e_real):
    mesh = plsc.VectorSubcoreMesh(core_axis_name="c", subcore_axis_name="s")
    return pl.kernel(
        functools.partial(_pass1_body, n_nodes, e_pad, e_real),
        out_type=[
            jax.ShapeDtypeStruct((e_pad,), jnp.float32),
            jax.ShapeDtypeStruct((NCORE, n_nodes, NLANE), jnp.float32),
        ],
        mesh=mesh,
        compiler_params=pltpu.CompilerParams(needs_layout_passes=False, use_tc_tiling_on_sc=False),
        scratch_types=[
            pltpu.VMEM((EB,), jnp.int32),
            pltpu.VMEM((EB,), jnp.int32),
            pltpu.VMEM((EB, 64), jnp.float32),
            pltpu.VMEM((EB, 64), jnp.float32),
            pltpu.VMEM((EB, HID), jnp.float32),
            pltpu.VMEM((EB, HID), jnp.float32),
            pltpu.VMEM((EB, NLANE), jnp.float32),
            pltpu.VMEM((EB,), jnp.float32),
            pltpu.VMEM((HID, NLANE), jnp.float32),
            pltpu.VMEM_SHARED((n_nodes, NLANE), jnp.float32),
            pltpu.SemaphoreType.DMA,
        ],
    )


def _node_halves(n_pad):
    if n_pad * 64 <= 1_700_000:
        return ((0, n_pad),)
    h0 = ((n_pad // 2 + 127) // 128) * 128
    return ((0, h0), (h0, n_pad - h0))


def _pass2_body(n_nodes, e_pad,
                xl0_hbm, xl1_hbm, src_hbm, dst_hbm, w_hbm, zeros_hbm,
                acc_hbm,
                idxs_v, idxl_v, rows_v, stage_v, w_v, acc_sp, sem):
    c = lax.axis_index("c")
    s = lax.axis_index("s")
    eps = e_pad // NSUB
    nblocks = eps // EB

    for (lo, cnt) in _node_halves(n_nodes):
        rows_per_sub = cnt // NSUB
        pltpu.sync_copy(zeros_hbm.at[pl.ds(s * rows_per_sub, rows_per_sub), :],
                        acc_sp.at[pl.ds(s * rows_per_sub, rows_per_sub), :])
        plsc.subcore_barrier()

        def block(t, _, lo=lo, cnt=cnt):
            base = s * eps + t * EB
            pltpu.sync_copy(src_hbm.at[pl.ds(base, EB)], idxs_v)
            pltpu.sync_copy(dst_hbm.at[pl.ds(base, EB)], idxl_v)
            pltpu.sync_copy(w_hbm.at[pl.ds(base, EB)], w_v)

            @pl.when(c == 0)
            def _g0():
                pltpu.async_copy(xl0_hbm.at[idxs_v], rows_v, sem).wait()

            @pl.when(c == 1)
            def _g1():
                pltpu.async_copy(xl1_hbm.at[idxs_v], rows_v, sem).wait()

            def group(g, _):
                sl = pl.ds(g * NLANE, NLANE)
                ei = lax.iota(jnp.int32, NLANE) + g * NLANE
                d = idxl_v[sl] - lo
                inb = (d >= 0) & (d < cnt)
                idxl_v[sl] = jnp.where(inb, d, 0)
                w = jnp.where(inb, w_v[sl], 0.0)
                for f in range(64):
                    fi = jnp.full((NLANE,), f, jnp.int32)
                    v = plsc.load_gather(rows_v, [ei, fi]) * w
                    plsc.store_scatter(stage_v, [ei, fi], v)
                return _

            lax.fori_loop(0, EB // NLANE, group, None)
            pltpu.sync_copy(stage_v, acc_sp.at[idxl_v], add=True)
            return _

        lax.fori_loop(0, nblocks, block, None)
        plsc.subcore_barrier()
        pltpu.sync_copy(
            acc_sp.at[pl.ds(s * rows_per_sub, rows_per_sub), :],
            acc_hbm.at[c].at[pl.ds(lo + s * rows_per_sub, rows_per_sub), :])
        plsc.subcore_barrier()


@functools.lru_cache(maxsize=None)
def _make_pass2(n_nodes, e_pad):
    mesh = plsc.VectorSubcoreMesh(core_axis_name="c", subcore_axis_name="s")
    return pl.kernel(
        functools.partial(_pass2_body, n_nodes, e_pad),
        out_type=jax.ShapeDtypeStruct((NCORE, n_nodes, 64), jnp.float32),
        mesh=mesh,
        compiler_params=pltpu.CompilerParams(needs_layout_passes=False, use_tc_tiling_on_sc=False),
        scratch_types=[
            pltpu.VMEM((EB,), jnp.int32),
            pltpu.VMEM((EB,), jnp.int32),
            pltpu.VMEM((EB, 64), jnp.float32),
            pltpu.VMEM((EB, 64), jnp.float32),
            pltpu.VMEM((EB,), jnp.float32),
            pltpu.VMEM_SHARED(
                (max(c for _, c in _node_halves(n_nodes)), 64), jnp.float32),
            pltpu.SemaphoreType.DMA,
        ],
    )


def _gat_layer(x, src, dst, ea_attr, p, i, n_nodes, e_real, zeros16, zeros64):
    e_pad = src.shape[0]
    n_pad = _pad_nodes(n_nodes)
    xl0, xl1, xr = _dense_node(x, p["Wl"][i], p["bl"][i], p["Wr"][i], p["br"][i])
    ea = _dense_edge(ea_attr, p["We"][i])
    att2 = jnp.broadcast_to(p["att"][i][:, None], (HID, NLANE))
    w, den = _make_pass1(n_pad, e_pad, e_real)(
        xl0, xl1, xr, ea, src, dst, att2, zeros16[:n_pad])
    acc = _make_pass2(n_pad, e_pad)(xl0, xl1, src, dst, w, zeros64[:n_pad])
    return _normalize(acc, den, p["bias"][i], n_nodes)


def _pad_edges(edge_index, edge_attr):
    e = edge_index.shape[1]
    e_pad = ((e + NW * EB - 1) // (NW * EB)) * (NW * EB)
    src = jnp.pad(edge_index[0], (0, e_pad - e))
    dst = jnp.pad(edge_index[1], (0, e_pad - e))
    ea = jnp.pad(edge_attr, ((0, e_pad - e), (0, 0)))
    return src, dst, ea, e


def kernel(x_lig, x_pro, lig_edge_index, lig_edge_attr, pro_edge_index,
           pro_edge_attr, inter_edge_index, inter_edge_attr,
           lig_params, pro_params, ip_params):
    n_lig = x_lig.shape[0]
    n_pro = x_pro.shape[0]
    n_int = n_lig + n_pro

    ls, ld, lea, l_e = _pad_edges(lig_edge_index, lig_edge_attr)
    ps, pd, pea, p_e = _pad_edges(pro_edge_index, pro_edge_attr)
    is_, id_, iea, i_e = _pad_edges(inter_edge_index, inter_edge_attr)

    zeros16 = jnp.zeros((_pad_nodes(n_int), NLANE), jnp.float32)
    zeros64 = jnp.zeros((_pad_nodes(n_int), 64), jnp.float32)

    for i in range(2):
        x_lig = _gat_layer(x_lig, ls, ld, lea, lig_params, i, n_lig, l_e,
                           zeros16, zeros64)
        x_pro = _gat_layer(x_pro, ps, pd, pea, pro_params, i, n_pro, p_e,
                           zeros16, zeros64)
        x_lp = jnp.concatenate([x_lig, x_pro], axis=0)
        x_lp = _gat_layer(x_lp, is_, id_, iea, ip_params, i, n_int, i_e,
                          zeros16, zeros64)
        x_lig = x_lp[:n_lig]
        x_pro = x_lp[n_lig:]
    return (x_pro, x_lig)


# speedup vs baseline: 1.4990x; 1.3320x over previous
import functools

import jax
import jax.numpy as jnp
from jax import lax
from jax.experimental import pallas as pl
from jax.experimental.pallas import tpu as pltpu
from jax.experimental.pallas import tpu_sc as plsc

HID = 128
NLANE = 16
NCORE = 2
NSUB = 16
NW = NCORE * NSUB
EB = 64
CH = 1024


def _dense_node_body(x_ref, wl_ref, bl_ref, wr_ref, br_ref,
                     xl0_ref, xl1_ref, xr_ref):
    x = x_ref[...]
    xl = jnp.dot(x, wl_ref[...], preferred_element_type=jnp.float32) + bl_ref[...]
    xr = jnp.dot(x, wr_ref[...], preferred_element_type=jnp.float32) + br_ref[...]
    xl0_ref[...] = xl[:, :64]
    xl1_ref[...] = xl[:, 64:]
    xr_ref[...] = xr


def _dense_node(x, wl, bl, wr, br):
    n = x.shape[0]
    br_blk = 256
    return pl.pallas_call(
        _dense_node_body,
        grid=(pl.cdiv(n, br_blk),),
        in_specs=[
            pl.BlockSpec((br_blk, HID), lambda i: (i, 0)),
            pl.BlockSpec((HID, HID), lambda i: (0, 0)),
            pl.BlockSpec((1, HID), lambda i: (0, 0)),
            pl.BlockSpec((HID, HID), lambda i: (0, 0)),
            pl.BlockSpec((1, HID), lambda i: (0, 0)),
        ],
        out_specs=[
            pl.BlockSpec((br_blk, 64), lambda i: (i, 0)),
            pl.BlockSpec((br_blk, 64), lambda i: (i, 0)),
            pl.BlockSpec((br_blk, HID), lambda i: (i, 0)),
        ],
        out_shape=[
            jax.ShapeDtypeStruct((n, 64), jnp.float32),
            jax.ShapeDtypeStruct((n, 64), jnp.float32),
            jax.ShapeDtypeStruct((n, HID), jnp.float32),
        ],
    )(x, wl, bl.reshape(1, HID), wr, br.reshape(1, HID))


def _dense_edge_body(a_ref, w_ref, o_ref):
    o_ref[...] = jnp.dot(a_ref[...], w_ref[...],
                         preferred_element_type=jnp.float32)


def _dense_edge(edge_attr, we):
    e, k = edge_attr.shape
    br_blk = 512
    return pl.pallas_call(
        _dense_edge_body,
        grid=(pl.cdiv(e, br_blk),),
        in_specs=[
            pl.BlockSpec((br_blk, k), lambda i: (i, 0)),
            pl.BlockSpec((k, HID), lambda i: (0, 0)),
        ],
        out_specs=pl.BlockSpec((br_blk, HID), lambda i: (i, 0)),
        out_shape=jax.ShapeDtypeStruct((e, HID), jnp.float32),
    )(edge_attr, we)


def _normalize_body(acc_ref, den_ref, bias_ref, out_ref):
    d = den_ref[0, :, 0:1] + den_ref[1, :, 0:1] + 1e-16
    out_ref[:, :64] = acc_ref[0] / d + bias_ref[:, :64]
    out_ref[:, 64:] = acc_ref[1] / d + bias_ref[:, 64:]


def _normalize(acc, den, bias, n):
    br_blk = 256
    return pl.pallas_call(
        _normalize_body,
        grid=(pl.cdiv(n, br_blk),),
        in_specs=[
            pl.BlockSpec((2, br_blk, 64), lambda i: (0, i, 0)),
            pl.BlockSpec((2, br_blk, NLANE), lambda i: (0, i, 0)),
            pl.BlockSpec((1, HID), lambda i: (0, 0)),
        ],
        out_specs=pl.BlockSpec((br_blk, HID), lambda i: (i, 0)),
        out_shape=jax.ShapeDtypeStruct((n, HID), jnp.float32),
    )(acc, den, bias.reshape(1, HID))


def _pad_nodes(n):
    return ((n + NSUB * 8 - 1) // (NSUB * 8)) * (NSUB * 8)


def _node_halves(n_pad):
    if n_pad * 64 <= 1_400_000:
        return ((0, n_pad),)
    h0 = ((n_pad // 2 + 127) // 128) * 128
    return ((0, h0), (h0, n_pad - h0))


def _pass1_body(n_nodes, e_pad, e_real,
                xl0_hbm, xl1_hbm, xr_hbm, ea_hbm, src_hbm, dst_hbm, att_hbm,
                zeros_hbm,
                w_hbm, den_hbm,
                idxs_ch, idxd_ch, bufl0, bufl1, bufd, bufe,
                stage, idxd_blk, w_v, att_v, den_sp,
                semg, semw, semsc):
    c = lax.axis_index("c")
    s = lax.axis_index("s")
    wid = s * NCORE + c
    epw = e_pad // NW
    nchunks = epw // CH
    nblk = CH // EB
    rows_per_sub = n_nodes // NSUB

    pltpu.sync_copy(zeros_hbm.at[pl.ds(s * rows_per_sub, rows_per_sub), :],
                    den_sp.at[pl.ds(s * rows_per_sub, rows_per_sub), :])
    pltpu.sync_copy(att_hbm, att_v)
    for b in range(2):
        pltpu.sync_copy(zeros_hbm.at[pl.ds(0, EB), :], stage[b])
    plsc.subcore_barrier()

    def g_descs(t, b):
        sl = pl.ds(t * EB, EB)
        return (
            pltpu.make_async_copy(xl0_hbm.at[idxs_ch.at[sl]], bufl0[b], semg[b]),
            pltpu.make_async_copy(xl1_hbm.at[idxs_ch.at[sl]], bufl1[b], semg[b]),
            pltpu.make_async_copy(xr_hbm.at[idxd_ch.at[sl]], bufd[b], semg[b]),
        )

    def chunk(ch, _):
        cbase = wid * epw + ch * CH
        pltpu.sync_copy(src_hbm.at[pl.ds(cbase, CH)], idxs_ch)
        pltpu.sync_copy(dst_hbm.at[pl.ds(cbase, CH)], idxd_ch)

        def fire(t, b):
            for dsc in g_descs(t, b):
                dsc.start()
            pltpu.make_async_copy(
                ea_hbm.at[pl.ds(cbase + t * EB, EB), :], bufe[b],
                semg[b]).start()

        fire(0, 0)
        fire(1, 1)

        def pair(tp, _):
            for b in range(2):
                t = 2 * tp + b
                for dsc in g_descs(t, b):
                    dsc.wait()
                pltpu.make_async_copy(
                    ea_hbm.at[pl.ds(cbase + t * EB, EB), :], bufe[b],
                    semg[b]).wait()

                @pl.when(t >= 2)
                def _():
                    pltpu.make_async_copy(
                        w_v[b], w_hbm.at[pl.ds(cbase + (t - 2) * EB, EB)],
                        semw[b]).wait()
                    pltpu.make_async_copy(
                        stage[b], den_sp.at[idxd_blk[b]], semsc[b]).wait()

                def group(g, _, b=b):
                    ei = lax.iota(jnp.int32, NLANE) + g * NLANE
                    idxd_blk[b][pl.ds(g * NLANE, NLANE)] = (
                        idxd_ch[pl.ds(t * EB + g * NLANE, NLANE)])
                    acc = jnp.zeros((NLANE,), jnp.float32)
                    for f in range(HID):
                        fi = jnp.full((NLANE,), f, jnp.int32)
                        if f < 64:
                            a = plsc.load_gather(bufl0[b], [ei, fi])
                        else:
                            a = plsc.load_gather(
                                bufl1[b],
                                [ei, jnp.full((NLANE,), f - 64, jnp.int32)])
                        v = (a + plsc.load_gather(bufd[b], [ei, fi])
                             + plsc.load_gather(bufe[b], [ei, fi]))
                        v = jnp.where(v >= 0, v, 0.2 * v)
                        acc = acc + att_v[f, :] * v
                    w = jnp.exp(acc)
                    gmask = (cbase + t * EB + ei) < e_real
                    w = jnp.where(gmask, w, 0.0)
                    w_v[b][pl.ds(g * NLANE, NLANE)] = w
                    plsc.store_scatter(stage[b],
                                       [ei, jnp.zeros((NLANE,), jnp.int32)], w)
                    return _

                lax.fori_loop(0, EB // NLANE, group, None)
                pltpu.make_async_copy(
                    w_v[b], w_hbm.at[pl.ds(cbase + t * EB, EB)],
                    semw[b]).start()
                pltpu.make_async_copy(
                    stage[b], den_sp.at[idxd_blk[b]], semsc[b]).start(add=True)

                @pl.when(t + 2 < nblk)
                def _():
                    fire(t + 2, b)
            return _

        lax.fori_loop(0, nblk // 2, pair, None)
        for b in range(2):
            t_last = nblk - 2 + b
            pltpu.make_async_copy(
                w_v[b], w_hbm.at[pl.ds(cbase + t_last * EB, EB)],
                semw[b]).wait()
            pltpu.make_async_copy(
                stage[b], den_sp.at[idxd_blk[b]], semsc[b]).wait()
        return _

    lax.fori_loop(0, nchunks, chunk, None)
    plsc.subcore_barrier()
    pltpu.sync_copy(den_sp.at[pl.ds(s * rows_per_sub, rows_per_sub), :],
                    den_hbm.at[c].at[pl.ds(s * rows_per_sub, rows_per_sub), :])


@functools.lru_cache(maxsize=None)
def _make_pass1(n_nodes, e_pad, e_real):
    mesh = plsc.VectorSubcoreMesh(core_axis_name="c", subcore_axis_name="s")
    return pl.kernel(
        functools.partial(_pass1_body, n_nodes, e_pad, e_real),
        out_type=[
            jax.ShapeDtypeStruct((e_pad,), jnp.float32),
            jax.ShapeDtypeStruct((NCORE, n_nodes, NLANE), jnp.float32),
        ],
        mesh=mesh,
        compiler_params=pltpu.CompilerParams(needs_layout_passes=False,
                                             use_tc_tiling_on_sc=False),
        scratch_types=[
            pltpu.VMEM((CH,), jnp.int32),
            pltpu.VMEM((CH,), jnp.int32),
            [pltpu.VMEM((EB, 64), jnp.float32) for _ in range(2)],
            [pltpu.VMEM((EB, 64), jnp.float32) for _ in range(2)],
            [pltpu.VMEM((EB, HID), jnp.float32) for _ in range(2)],
            [pltpu.VMEM((EB, HID), jnp.float32) for _ in range(2)],
            [pltpu.VMEM((EB, NLANE), jnp.float32) for _ in range(2)],
            [pltpu.VMEM((EB,), jnp.int32) for _ in range(2)],
            [pltpu.VMEM((EB,), jnp.float32) for _ in range(2)],
            pltpu.VMEM((HID, NLANE), jnp.float32),
            pltpu.VMEM_SHARED((n_nodes, NLANE), jnp.float32),
            [pltpu.SemaphoreType.DMA for _ in range(2)],
            [pltpu.SemaphoreType.DMA for _ in range(2)],
            [pltpu.SemaphoreType.DMA for _ in range(2)],
        ],
    )


def _pass2_body(n_nodes, e_pad,
                xl0_hbm, xl1_hbm, src_hbm, dst_hbm, w_hbm, zeros_hbm,
                acc_hbm,
                idxs_ch, idxd_ch, w_ch, rows, stage, idxl, acc_sp,
                semg, semsc):
    c = lax.axis_index("c")
    s = lax.axis_index("s")
    eps = e_pad // NSUB
    nchunks = eps // CH
    nblk = CH // EB

    def fire(t, b):
        sl = pl.ds(t * EB, EB)

        @pl.when(c == 0)
        def _():
            pltpu.make_async_copy(
                xl0_hbm.at[idxs_ch.at[sl]], rows[b], semg[b]).start()

        @pl.when(c == 1)
        def _():
            pltpu.make_async_copy(
                xl1_hbm.at[idxs_ch.at[sl]], rows[b], semg[b]).start()

    for (lo, cnt) in _node_halves(n_nodes):
        rows_per_sub = cnt // NSUB
        pltpu.sync_copy(zeros_hbm.at[pl.ds(s * rows_per_sub, rows_per_sub), :],
                        acc_sp.at[pl.ds(s * rows_per_sub, rows_per_sub), :])
        plsc.subcore_barrier()

        def chunk(ch, _, lo=lo, cnt=cnt):
            cbase = s * eps + ch * CH
            pltpu.sync_copy(src_hbm.at[pl.ds(cbase, CH)], idxs_ch)
            pltpu.sync_copy(dst_hbm.at[pl.ds(cbase, CH)], idxd_ch)
            pltpu.sync_copy(w_hbm.at[pl.ds(cbase, CH)], w_ch)

            fire(0, 0)
            fire(1, 1)

            def pair(tp, _):
                for b in range(2):
                    t = 2 * tp + b
                    pltpu.make_async_copy(
                        xl0_hbm.at[idxs_ch.at[pl.ds(t * EB, EB)]], rows[b],
                        semg[b]).wait()

                    @pl.when(t >= 2)
                    def _():
                        pltpu.make_async_copy(
                            stage[b], acc_sp.at[idxl[b]], semsc[b]).wait()

                    def group(g, _, b=b):
                        sl16 = pl.ds(g * NLANE, NLANE)
                        ei = lax.iota(jnp.int32, NLANE) + g * NLANE
                        d = idxd_ch[pl.ds(t * EB + g * NLANE, NLANE)] - lo
                        inb = (d >= 0) & (d < cnt)
                        idxl[b][sl16] = jnp.where(inb, d, 0)
                        w = jnp.where(
                            inb, w_ch[pl.ds(t * EB + g * NLANE, NLANE)], 0.0)
                        for f in range(64):
                            fi = jnp.full((NLANE,), f, jnp.int32)
                            v = plsc.load_gather(rows[b], [ei, fi]) * w
                            plsc.store_scatter(stage[b], [ei, fi], v)
                        return _

                    lax.fori_loop(0, EB // NLANE, group, None)
                    pltpu.make_async_copy(
                        stage[b], acc_sp.at[idxl[b]], semsc[b]).start(add=True)

                    @pl.when(t + 2 < nblk)
                    def _():
                        fire(t + 2, b)
                return _

            lax.fori_loop(0, nblk // 2, pair, None)
            for b in range(2):
                pltpu.make_async_copy(
                    stage[b], acc_sp.at[idxl[b]], semsc[b]).wait()
            return _

        lax.fori_loop(0, nchunks, chunk, None)
        plsc.subcore_barrier()
        pltpu.sync_copy(
            acc_sp.at[pl.ds(s * rows_per_sub, rows_per_sub), :],
            acc_hbm.at[c].at[pl.ds(lo + s * rows_per_sub, rows_per_sub), :])
        plsc.subcore_barrier()


@functools.lru_cache(maxsize=None)
def _make_pass2(n_nodes, e_pad):
    mesh = plsc.VectorSubcoreMesh(core_axis_name="c", subcore_axis_name="s")
    return pl.kernel(
        functools.partial(_pass2_body, n_nodes, e_pad),
        out_type=jax.ShapeDtypeStruct((NCORE, n_nodes, 64), jnp.float32),
        mesh=mesh,
        compiler_params=pltpu.CompilerParams(needs_layout_passes=False,
                                             use_tc_tiling_on_sc=False),
        scratch_types=[
            pltpu.VMEM((CH,), jnp.int32),
            pltpu.VMEM((CH,), jnp.int32),
            pltpu.VMEM((CH,), jnp.float32),
            [pltpu.VMEM((EB, 64), jnp.float32) for _ in range(2)],
            [pltpu.VMEM((EB, 64), jnp.float32) for _ in range(2)],
            [pltpu.VMEM((EB,), jnp.int32) for _ in range(2)],
            pltpu.VMEM_SHARED(
                (max(cc for _, cc in _node_halves(n_nodes)), 64), jnp.float32),
            [pltpu.SemaphoreType.DMA for _ in range(2)],
            [pltpu.SemaphoreType.DMA for _ in range(2)],
        ],
    )


def _gat_layer(x, src, dst, ea_attr, p, i, n_nodes, e_real, zeros16, zeros64):
    e_pad = src.shape[0]
    n_pad = _pad_nodes(n_nodes)
    xl0, xl1, xr = _dense_node(x, p["Wl"][i], p["bl"][i], p["Wr"][i], p["br"][i])
    ea = _dense_edge(ea_attr, p["We"][i])
    att2 = jnp.broadcast_to(p["att"][i][:, None], (HID, NLANE))
    w, den = _make_pass1(n_pad, e_pad, e_real)(
        xl0, xl1, xr, ea, src, dst, att2, zeros16[:n_pad])
    acc = _make_pass2(n_pad, e_pad)(xl0, xl1, src, dst, w, zeros64[:n_pad])
    return _normalize(acc, den, p["bias"][i], n_nodes)


def _pad_edges(edge_index, edge_attr):
    e = edge_index.shape[1]
    m = NW * CH
    e_pad = ((e + m - 1) // m) * m
    src = jnp.pad(edge_index[0], (0, e_pad - e))
    dst = jnp.pad(edge_index[1], (0, e_pad - e))
    ea = jnp.pad(edge_attr, ((0, e_pad - e), (0, 0)))
    return src, dst, ea, e


def kernel(x_lig, x_pro, lig_edge_index, lig_edge_attr, pro_edge_index,
           pro_edge_attr, inter_edge_index, inter_edge_attr,
           lig_params, pro_params, ip_params):
    n_lig = x_lig.shape[0]
    n_pro = x_pro.shape[0]
    n_int = n_lig + n_pro

    ls, ld, lea, l_e = _pad_edges(lig_edge_index, lig_edge_attr)
    ps, pd, pea, p_e = _pad_edges(pro_edge_index, pro_edge_attr)
    is_, id_, iea, i_e = _pad_edges(inter_edge_index, inter_edge_attr)

    zeros16 = jnp.zeros((_pad_nodes(n_int), NLANE), jnp.float32)
    zeros64 = jnp.zeros((_pad_nodes(n_int), 64), jnp.float32)

    for i in range(2):
        x_lig = _gat_layer(x_lig, ls, ld, lea, lig_params, i, n_lig, l_e,
                           zeros16, zeros64)
        x_pro = _gat_layer(x_pro, ps, pd, pea, pro_params, i, n_pro, p_e,
                           zeros16, zeros64)
        x_lp = jnp.concatenate([x_lig, x_pro], axis=0)
        x_lp = _gat_layer(x_lp, is_, id_, iea, ip_params, i, n_int, i_e,
                          zeros16, zeros64)
        x_lig = x_lp[:n_lig]
        x_pro = x_lp[n_lig:]
    return (x_pro, x_lig)
